# baseline (speedup 1.0000x reference)
import numpy as np
import jax
import jax.numpy as jnp
from jax.experimental import pallas as pl
from jax.experimental.pallas import tpu as pltpu
from jax.experimental.pallas import tpu_sc as plsc

N = 4096
D = 128
TAU = 1.0
SIGMA = 1.2
R_EMIS = 0.5
LOG2PI = float(np.log(2.0 * np.pi))

BLK = 256
GRID = N // BLK


def _np_rotl(x, d):
    return ((x << np.uint32(d)) | (x >> np.uint32(32 - d))).astype(np.uint32)


def _np_threefry2x32(k1, k2, x0, x1):
    x0 = x0.astype(np.uint32)
    x1 = x1.astype(np.uint32)
    ks0 = np.uint32(k1)
    ks1 = np.uint32(k2)
    ks2 = np.uint32(0x1BD11BDA) ^ ks0 ^ ks1
    ks = [ks0, ks1, ks2]
    rots = [(13, 15, 26, 6), (17, 29, 16, 24)]
    x0 = (x0 + ks0).astype(np.uint32)
    x1 = (x1 + ks1).astype(np.uint32)
    for i in range(5):
        for r in rots[i % 2]:
            x0 = (x0 + x1).astype(np.uint32)
            x1 = _np_rotl(x1, r)
            x1 = x1 ^ x0
        x0 = (x0 + ks[(i + 1) % 3]).astype(np.uint32)
        x1 = (x1 + ks[(i + 2) % 3] + np.uint32(i + 1)).astype(np.uint32)
    return x0, x1


_S0, _S1 = _np_threefry2x32(0, 1, np.zeros(2, np.uint32), np.arange(2, dtype=np.uint32))
RK0, RK1 = int(_S0[0]), int(_S1[0])
PK0, PK1 = int(_S0[1]), int(_S1[1])

U_MIN = np.float32(1e-12)
U_SCALE = np.float32(1.0) - np.float32(1e-12)
N_LO = np.float32(np.nextafter(np.float32(-1.0), np.float32(0.0)))
N_SCALE = np.float32(1.0) - N_LO
SQRT2 = np.float32(np.sqrt(2.0))


def _threefry(k1, k2, x1):
    ks0 = jnp.uint32(k1)
    ks1 = jnp.uint32(k2)
    ks2 = jnp.uint32(np.uint32(0x1BD11BDA) ^ np.uint32(k1) ^ np.uint32(k2))
    ks = (ks0, ks1, ks2)
    rots = ((13, 15, 26, 6), (17, 29, 16, 24))
    x0 = jnp.full(x1.shape, ks0, jnp.uint32)
    x1 = x1 + ks1
    for i in range(5):
        for r in rots[i % 2]:
            x0 = x0 + x1
            x1 = (x1 << r) | (x1 >> (32 - r))
            x1 = x1 ^ x0
        x0 = x0 + ks[(i + 1) % 3]
        x1 = x1 + ks[(i + 2) % 3] + jnp.uint32(i + 1)
    return x0 ^ x1


def _bits_to_f01(bits):
    fb = (bits >> 9) | jnp.uint32(0x3F800000)
    return jax.lax.bitcast_convert_type(fb, jnp.float32) - jnp.float32(1.0)


def _erfinv(x):
    w = -jnp.log((jnp.float32(1.0) - x) * (jnp.float32(1.0) + x))
    ws = w - jnp.float32(2.5)
    p = jnp.float32(2.81022636e-08)
    for c in (3.43273939e-07, -3.5233877e-06, -4.39150654e-06, 0.00021858087,
              -0.00125372503, -0.00417768164, 0.246640727, 1.50140941):
        p = jnp.float32(c) + p * ws
    wl = jnp.sqrt(w) - jnp.float32(3.0)
    q = jnp.float32(-0.000200214257)
    for c in (0.000100950558, 0.00134934322, -0.00367342844, 0.00573950773,
              -0.0076224613, 0.00943887047, 1.00167406, 2.83297682):
        q = jnp.float32(c) + q * wl
    return jnp.where(w < jnp.float32(5.0), p, q) * x


CH = 8
CW = 512
NCC = N // CW


def _argmax_kernel(lw_ref, ixo_ref, ess_ref):
    i = pl.program_id(0)
    r0 = i * BLK

    lw = lw_ref[:]
    m = jnp.max(lw)
    t = jnp.exp(lw - m)
    s1 = jnp.sum(t)
    s2 = jnp.sum(t * t)
    ess = s1 * s1 / (s2 * jnp.float32(N))
    ess_ref[...] = jnp.reshape(ess, (1, 1, 1))
    resample = ess < jnp.float32(0.5)

    row_s = jax.lax.broadcasted_iota(jnp.int32, (CH, CW), 0)
    col_s = jax.lax.broadcasted_iota(jnp.int32, (CH, CW), 1)
    pat = ((row_s << 12) | col_s).astype(jnp.uint32)
    rowid = jax.lax.broadcasted_iota(jnp.int32, (CH, 1), 0)

    def row_chunk(rc, carry):
        rbase = ((r0 + rc * CH) << 12).astype(jnp.uint32)
        M = jnp.full((CH, CW), -jnp.inf, jnp.float32)
        IDX = jnp.zeros((CH, CW), jnp.int32)
        for cc in range(NCC):
            bits = _threefry(RK0, RK1, pat + (rbase + jnp.uint32(cc * CW)))
            f01 = _bits_to_f01(bits)
            u = jnp.maximum(U_MIN, f01 + U_MIN)
            vals = lw_ref[pl.ds(cc * CW, CW)][None, :] + (-jnp.log(-jnp.log(u)))
            upd = vals > M
            M = jnp.where(upd, vals, M)
            IDX = jnp.where(upd, cc, IDX)
        rowV = jnp.max(M, axis=1, keepdims=True)
        jcand = (IDX << 9) | col_s
        ix = jnp.min(jnp.where(M == rowV, jcand, N), axis=1, keepdims=True)
        ix_final = jnp.where(resample, ix, rowid + (r0 + rc * CH))
        ixo_ref[pl.ds(rc * CH, CH), :] = ix_final
        return carry

    jax.lax.fori_loop(0, BLK // CH, row_chunk, 0, unroll=4)


_SC_INFO = plsc.get_sparse_core_info()
_NW = _SC_INFO.num_cores * _SC_INFO.num_subcores
_BPW = N // _NW


def _sc_gather_body(p_hbm, idx_hbm, out_hbm, idx_v, rows_v, sem):
    wid = (jax.lax.axis_index("s") * _SC_INFO.num_cores
           + jax.lax.axis_index("c"))
    base = wid * _BPW
    pltpu.sync_copy(idx_hbm.at[pl.ds(base, _BPW)], idx_v)
    pltpu.async_copy(p_hbm.at[idx_v], rows_v, sem).wait()
    pltpu.sync_copy(rows_v, out_hbm.at[pl.ds(base, _BPW)])


def _sc_gather(particles, idx):
    mesh = plsc.VectorSubcoreMesh(core_axis_name="c", subcore_axis_name="s")
    return pl.kernel(
        _sc_gather_body,
        mesh=mesh,
        out_type=jax.ShapeDtypeStruct((N, D), jnp.float32),
        scratch_types=[
            pltpu.VMEM((_BPW,), jnp.int32),
            pltpu.VMEM((_BPW, D), jnp.float32),
            pltpu.SemaphoreType.DMA,
        ],
    )(particles, idx)


CBLK = 1024
CGRID = N // CBLK


def _epilogue_kernel(lw_ref, pr_ref, obs_ref, logw_ref, next_ref):
    i = pl.program_id(0)
    r0 = i * CBLK

    lw = lw_ref[:]
    m = jnp.max(lw)
    t = jnp.exp(lw - m)
    s1 = jnp.sum(t)
    s2 = jnp.sum(t * t)
    ess = s1 * s1 / (s2 * jnp.float32(N))
    resample = ess < jnp.float32(0.5)

    ctr2 = ((r0 + jax.lax.broadcasted_iota(jnp.int32, (CBLK, D), 0)) * D
            + jax.lax.broadcasted_iota(jnp.int32, (CBLK, D), 1)).astype(jnp.uint32)
    f2 = _bits_to_f01(_threefry(PK0, PK1, ctr2))
    u2 = jnp.maximum(N_LO, f2 * N_SCALE + N_LO)
    seps = jnp.float32(SIGMA) * (SQRT2 * _erfinv(u2))

    pr = pr_ref[...]
    nxt = pr + seps
    next_ref[...] = nxt

    diff = nxt - pr
    obs = obs_ref[:]
    dobs = obs[None, :] - nxt
    half_d = jnp.float32(0.5 * D)
    trans = (-0.5 * jnp.sum((diff / jnp.float32(TAU)) ** 2, axis=1)
             - jnp.float32(D * np.log(TAU)) - half_d * jnp.float32(LOG2PI))
    emis = (-0.5 * jnp.sum((dobs / jnp.float32(R_EMIS)) ** 2, axis=1)
            - jnp.float32(D * np.log(R_EMIS)) - half_d * jnp.float32(LOG2PI))
    prop = (-0.5 * jnp.sum((diff / jnp.float32(SIGMA)) ** 2, axis=1)
            - jnp.float32(D * np.log(SIGMA)) - half_d * jnp.float32(LOG2PI))

    lw_blk = lw_ref[pl.ds(r0, CBLK)]
    base = jnp.where(resample, jnp.float32(0.0), lw_blk)
    logw_ref[...] = base + trans + emis - prop


def kernel(log_weights, particles, observation):
    ixo, ess = pl.pallas_call(
        _argmax_kernel,
        grid=(GRID,),
        in_specs=[pl.BlockSpec((N,), lambda i: (0,))],
        out_specs=[
            pl.BlockSpec((BLK, 1), lambda i: (i, 0)),
            pl.BlockSpec((1, 1, 1), lambda i: (i, 0, 0)),
        ],
        out_shape=[
            jax.ShapeDtypeStruct((N, 1), jnp.int32),
            jax.ShapeDtypeStruct((GRID, 1, 1), jnp.float32),
        ],
        compiler_params=pltpu.CompilerParams(
            dimension_semantics=("parallel",)),
    )(log_weights)

    pr = _sc_gather(particles, ixo.reshape(N))

    logw, nxt = pl.pallas_call(
        _epilogue_kernel,
        grid=(CGRID,),
        in_specs=[
            pl.BlockSpec((N,), lambda i: (0,)),
            pl.BlockSpec((CBLK, D), lambda i: (i, 0)),
            pl.BlockSpec((D,), lambda i: (0,)),
        ],
        out_specs=[
            pl.BlockSpec((CBLK,), lambda i: (i,)),
            pl.BlockSpec((CBLK, D), lambda i: (i, 0)),
        ],
        out_shape=[
            jax.ShapeDtypeStruct((N,), jnp.float32),
            jax.ShapeDtypeStruct((N, D), jnp.float32),
        ],
        compiler_params=pltpu.CompilerParams(
            dimension_semantics=("parallel",)),
    )(log_weights, pr, observation)

    return logw, nxt, ess[0, 0, 0]


# speedup vs baseline: 1.0570x; 1.0570x over previous
import numpy as np
import jax
import jax.numpy as jnp
from jax.experimental import pallas as pl
from jax.experimental.pallas import tpu as pltpu
from jax.experimental.pallas import tpu_sc as plsc

N = 4096
D = 128
TAU = 1.0
SIGMA = 1.2
R_EMIS = 0.5
LOG2PI = float(np.log(2.0 * np.pi))

BLK = 512
GRID = N // BLK


def _np_rotl(x, d):
    return ((x << np.uint32(d)) | (x >> np.uint32(32 - d))).astype(np.uint32)


def _np_threefry2x32(k1, k2, x0, x1):
    x0 = x0.astype(np.uint32)
    x1 = x1.astype(np.uint32)
    ks0 = np.uint32(k1)
    ks1 = np.uint32(k2)
    ks2 = np.uint32(0x1BD11BDA) ^ ks0 ^ ks1
    ks = [ks0, ks1, ks2]
    rots = [(13, 15, 26, 6), (17, 29, 16, 24)]
    x0 = (x0 + ks0).astype(np.uint32)
    x1 = (x1 + ks1).astype(np.uint32)
    for i in range(5):
        for r in rots[i % 2]:
            x0 = (x0 + x1).astype(np.uint32)
            x1 = _np_rotl(x1, r)
            x1 = x1 ^ x0
        x0 = (x0 + ks[(i + 1) % 3]).astype(np.uint32)
        x1 = (x1 + ks[(i + 2) % 3] + np.uint32(i + 1)).astype(np.uint32)
    return x0, x1


_S0, _S1 = _np_threefry2x32(0, 1, np.zeros(2, np.uint32), np.arange(2, dtype=np.uint32))
RK0, RK1 = int(_S0[0]), int(_S1[0])
PK0, PK1 = int(_S0[1]), int(_S1[1])

U_MIN = np.float32(1e-12)
U_SCALE = np.float32(1.0) - np.float32(1e-12)
N_LO = np.float32(np.nextafter(np.float32(-1.0), np.float32(0.0)))
N_SCALE = np.float32(1.0) - N_LO
SQRT2 = np.float32(np.sqrt(2.0))


def _threefry(k1, k2, x1, pre_keyed=False):
    ks0 = jnp.uint32(k1)
    ks1 = jnp.uint32(k2)
    ks2 = jnp.uint32(np.uint32(0x1BD11BDA) ^ np.uint32(k1) ^ np.uint32(k2))
    ks = (ks0, ks1, ks2)
    rots = ((13, 15, 26, 6), (17, 29, 16, 24))
    x0 = jnp.full(x1.shape, ks0, jnp.uint32)
    if not pre_keyed:
        x1 = x1 + ks1
    for i in range(5):
        for r in rots[i % 2]:
            x0 = x0 + x1
            x1 = (x1 << r) | (x1 >> (32 - r))
            x1 = x1 ^ x0
        x0 = x0 + ks[(i + 1) % 3]
        x1 = x1 + ks[(i + 2) % 3] + jnp.uint32(i + 1)
    return x0 ^ x1


def _bits_to_f01(bits):
    fb = (bits >> 9) | jnp.uint32(0x3F800000)
    return jax.lax.bitcast_convert_type(fb, jnp.float32) - jnp.float32(1.0)


def _erfinv(x):
    w = -jnp.log((jnp.float32(1.0) - x) * (jnp.float32(1.0) + x))
    ws = w - jnp.float32(2.5)
    p = jnp.float32(2.81022636e-08)
    for c in (3.43273939e-07, -3.5233877e-06, -4.39150654e-06, 0.00021858087,
              -0.00125372503, -0.00417768164, 0.246640727, 1.50140941):
        p = jnp.float32(c) + p * ws
    wl = jnp.sqrt(w) - jnp.float32(3.0)
    q = jnp.float32(-0.000200214257)
    for c in (0.000100950558, 0.00134934322, -0.00367342844, 0.00573950773,
              -0.0076224613, 0.00943887047, 1.00167406, 2.83297682):
        q = jnp.float32(c) + q * wl
    return jnp.where(w < jnp.float32(5.0), p, q) * x


CH = 8
CW = 512
NCC = N // CW


def _argmax_kernel(lw_ref, ixo_ref, ess_ref):
    i = pl.program_id(0)
    r0 = i * BLK

    lw = lw_ref[:]
    m = jnp.max(lw)
    t = jnp.exp(lw - m)
    s1 = jnp.sum(t)
    s2 = jnp.sum(t * t)
    ess = s1 * s1 / (s2 * jnp.float32(N))
    ess_ref[...] = jnp.reshape(ess, (1, 1, 1))
    resample = ess < jnp.float32(0.5)

    row_s = jax.lax.broadcasted_iota(jnp.int32, (CH, CW), 0)
    col_s = jax.lax.broadcasted_iota(jnp.int32, (CH, CW), 1)
    pat = ((row_s << 12) | col_s).astype(jnp.uint32) + jnp.uint32(RK1)
    rowid = jax.lax.broadcasted_iota(jnp.int32, (CH, 1), 0)

    def row_chunk(rc, carry):
        rbase = ((r0 + rc * CH) << 12).astype(jnp.uint32)
        M = jnp.full((CH, CW), -jnp.inf, jnp.float32)
        IDX = jnp.zeros((CH, CW), jnp.int32)
        for cc in range(NCC):
            bits = _threefry(RK0, RK1, pat + (rbase + jnp.uint32(cc * CW)),
                             pre_keyed=True)
            f01 = _bits_to_f01(bits)
            u = jnp.maximum(U_MIN, f01 + U_MIN)
            vals = lw_ref[pl.ds(cc * CW, CW)][None, :] + (-jnp.log(-jnp.log(u)))
            upd = vals > M
            M = jnp.where(upd, vals, M)
            IDX = jnp.where(upd, cc, IDX)
        rowV = jnp.max(M, axis=1, keepdims=True)
        jcand = (IDX << 9) | col_s
        ix = jnp.min(jnp.where(M == rowV, jcand, N), axis=1, keepdims=True)
        ix_final = jnp.where(resample, ix, rowid + (r0 + rc * CH))
        ixo_ref[pl.ds(rc * CH, CH), :] = ix_final
        return carry

    jax.lax.fori_loop(0, BLK // CH, row_chunk, 0, unroll=8)


_SC_INFO = plsc.get_sparse_core_info()
_NW = _SC_INFO.num_cores * _SC_INFO.num_subcores
_BPW = N // _NW


def _sc_gather_body(p_hbm, idx_hbm, out_hbm, idx_v, rows_v, sem):
    wid = (jax.lax.axis_index("s") * _SC_INFO.num_cores
           + jax.lax.axis_index("c"))
    base = wid * _BPW
    pltpu.sync_copy(idx_hbm.at[pl.ds(base, _BPW)], idx_v)
    pltpu.async_copy(p_hbm.at[idx_v], rows_v, sem).wait()
    pltpu.sync_copy(rows_v, out_hbm.at[pl.ds(base, _BPW)])


def _sc_gather(particles, idx):
    mesh = plsc.VectorSubcoreMesh(core_axis_name="c", subcore_axis_name="s")
    return pl.kernel(
        _sc_gather_body,
        mesh=mesh,
        out_type=jax.ShapeDtypeStruct((N, D), jnp.float32),
        scratch_types=[
            pltpu.VMEM((_BPW,), jnp.int32),
            pltpu.VMEM((_BPW, D), jnp.float32),
            pltpu.SemaphoreType.DMA,
        ],
    )(particles, idx)


CBLK = 1024
CGRID = N // CBLK


def _epilogue_kernel(lw_ref, pr_ref, obs_ref, logw_ref, next_ref):
    i = pl.program_id(0)
    r0 = i * CBLK

    lw = lw_ref[:]
    m = jnp.max(lw)
    t = jnp.exp(lw - m)
    s1 = jnp.sum(t)
    s2 = jnp.sum(t * t)
    ess = s1 * s1 / (s2 * jnp.float32(N))
    resample = ess < jnp.float32(0.5)

    ctr2 = ((r0 + jax.lax.broadcasted_iota(jnp.int32, (CBLK, D), 0)) * D
            + jax.lax.broadcasted_iota(jnp.int32, (CBLK, D), 1)).astype(jnp.uint32)
    f2 = _bits_to_f01(_threefry(PK0, PK1, ctr2))
    u2 = jnp.maximum(N_LO, f2 * N_SCALE + N_LO)
    seps = jnp.float32(SIGMA) * (SQRT2 * _erfinv(u2))

    pr = pr_ref[...]
    nxt = pr + seps
    next_ref[...] = nxt

    diff = nxt - pr
    obs = obs_ref[:]
    dobs = obs[None, :] - nxt
    half_d = jnp.float32(0.5 * D)
    trans = (-0.5 * jnp.sum((diff / jnp.float32(TAU)) ** 2, axis=1)
             - jnp.float32(D * np.log(TAU)) - half_d * jnp.float32(LOG2PI))
    emis = (-0.5 * jnp.sum((dobs / jnp.float32(R_EMIS)) ** 2, axis=1)
            - jnp.float32(D * np.log(R_EMIS)) - half_d * jnp.float32(LOG2PI))
    prop = (-0.5 * jnp.sum((diff / jnp.float32(SIGMA)) ** 2, axis=1)
            - jnp.float32(D * np.log(SIGMA)) - half_d * jnp.float32(LOG2PI))

    lw_blk = lw_ref[pl.ds(r0, CBLK)]
    base = jnp.where(resample, jnp.float32(0.0), lw_blk)
    logw_ref[...] = base + trans + emis - prop


def kernel(log_weights, particles, observation):
    ixo, ess = pl.pallas_call(
        _argmax_kernel,
        grid=(GRID,),
        in_specs=[pl.BlockSpec((N,), lambda i: (0,))],
        out_specs=[
            pl.BlockSpec((BLK, 1), lambda i: (i, 0)),
            pl.BlockSpec((1, 1, 1), lambda i: (i, 0, 0)),
        ],
        out_shape=[
            jax.ShapeDtypeStruct((N, 1), jnp.int32),
            jax.ShapeDtypeStruct((GRID, 1, 1), jnp.float32),
        ],
        compiler_params=pltpu.CompilerParams(
            dimension_semantics=("parallel",)),
    )(log_weights)

    pr = _sc_gather(particles, ixo.reshape(N))

    logw, nxt = pl.pallas_call(
        _epilogue_kernel,
        grid=(CGRID,),
        in_specs=[
            pl.BlockSpec((N,), lambda i: (0,)),
            pl.BlockSpec((CBLK, D), lambda i: (i, 0)),
            pl.BlockSpec((D,), lambda i: (0,)),
        ],
        out_specs=[
            pl.BlockSpec((CBLK,), lambda i: (i,)),
            pl.BlockSpec((CBLK, D), lambda i: (i, 0)),
        ],
        out_shape=[
            jax.ShapeDtypeStruct((N,), jnp.float32),
            jax.ShapeDtypeStruct((N, D), jnp.float32),
        ],
        compiler_params=pltpu.CompilerParams(
            dimension_semantics=("parallel",)),
    )(log_weights, pr, observation)

    return logw, nxt, ess[0, 0, 0]


# speedup vs baseline: 1.0723x; 1.0144x over previous
import numpy as np
import jax
import jax.numpy as jnp
from jax.experimental import pallas as pl
from jax.experimental.pallas import tpu as pltpu
from jax.experimental.pallas import tpu_sc as plsc

N = 4096
D = 128
TAU = 1.0
SIGMA = 1.2
R_EMIS = 0.5
LOG2PI = float(np.log(2.0 * np.pi))

BLK = 512
GRID = N // BLK


def _np_rotl(x, d):
    return ((x << np.uint32(d)) | (x >> np.uint32(32 - d))).astype(np.uint32)


def _np_threefry2x32(k1, k2, x0, x1):
    x0 = x0.astype(np.uint32)
    x1 = x1.astype(np.uint32)
    ks0 = np.uint32(k1)
    ks1 = np.uint32(k2)
    ks2 = np.uint32(0x1BD11BDA) ^ ks0 ^ ks1
    ks = [ks0, ks1, ks2]
    rots = [(13, 15, 26, 6), (17, 29, 16, 24)]
    x0 = (x0 + ks0).astype(np.uint32)
    x1 = (x1 + ks1).astype(np.uint32)
    for i in range(5):
        for r in rots[i % 2]:
            x0 = (x0 + x1).astype(np.uint32)
            x1 = _np_rotl(x1, r)
            x1 = x1 ^ x0
        x0 = (x0 + ks[(i + 1) % 3]).astype(np.uint32)
        x1 = (x1 + ks[(i + 2) % 3] + np.uint32(i + 1)).astype(np.uint32)
    return x0, x1


_S0, _S1 = _np_threefry2x32(0, 1, np.zeros(2, np.uint32), np.arange(2, dtype=np.uint32))
RK0, RK1 = int(_S0[0]), int(_S1[0])
PK0, PK1 = int(_S0[1]), int(_S1[1])

U_MIN = np.float32(1e-12)
U_SCALE = np.float32(1.0) - np.float32(1e-12)
N_LO = np.float32(np.nextafter(np.float32(-1.0), np.float32(0.0)))
N_SCALE = np.float32(1.0) - N_LO
SQRT2 = np.float32(np.sqrt(2.0))


def _threefry(k1, k2, x1, pre_keyed=False):
    ks0 = jnp.uint32(k1)
    ks1 = jnp.uint32(k2)
    ks2 = jnp.uint32(np.uint32(0x1BD11BDA) ^ np.uint32(k1) ^ np.uint32(k2))
    ks = (ks0, ks1, ks2)
    rots = ((13, 15, 26, 6), (17, 29, 16, 24))
    x0 = jnp.full(x1.shape, ks0, jnp.uint32)
    if not pre_keyed:
        x1 = x1 + ks1
    for i in range(5):
        for r in rots[i % 2]:
            x0 = x0 + x1
            x1 = (x1 << r) | (x1 >> (32 - r))
            x1 = x1 ^ x0
        x0 = x0 + ks[(i + 1) % 3]
        x1 = x1 + ks[(i + 2) % 3] + jnp.uint32(i + 1)
    return x0 ^ x1


def _bits_to_f01(bits):
    fb = (bits >> 9) | jnp.uint32(0x3F800000)
    return jax.lax.bitcast_convert_type(fb, jnp.float32) - jnp.float32(1.0)


def _erfinv(x):
    w = -jnp.log((jnp.float32(1.0) - x) * (jnp.float32(1.0) + x))
    ws = w - jnp.float32(2.5)
    p = jnp.float32(2.81022636e-08)
    for c in (3.43273939e-07, -3.5233877e-06, -4.39150654e-06, 0.00021858087,
              -0.00125372503, -0.00417768164, 0.246640727, 1.50140941):
        p = jnp.float32(c) + p * ws
    wl = jnp.sqrt(w) - jnp.float32(3.0)
    q = jnp.float32(-0.000200214257)
    for c in (0.000100950558, 0.00134934322, -0.00367342844, 0.00573950773,
              -0.0076224613, 0.00943887047, 1.00167406, 2.83297682):
        q = jnp.float32(c) + q * wl
    return jnp.where(w < jnp.float32(5.0), p, q) * x


CH = 8
CW = 512
NCC = N // CW


def _argmax_kernel(lw_ref, ixo_ref, ess_ref):
    i = pl.program_id(0)
    r0 = i * BLK

    lw = lw_ref[:]
    m = jnp.max(lw)
    t = jnp.exp(lw - m)
    s1 = jnp.sum(t)
    s2 = jnp.sum(t * t)
    ess = s1 * s1 / (s2 * jnp.float32(N))
    ess_ref[...] = jnp.reshape(ess, (1, 1, 1))
    resample = ess < jnp.float32(0.5)

    row_s = jax.lax.broadcasted_iota(jnp.int32, (CH, CW), 0)
    col_s = jax.lax.broadcasted_iota(jnp.int32, (CH, CW), 1)
    pat = ((row_s << 12) | col_s).astype(jnp.uint32) + jnp.uint32(RK1)
    rowid = jax.lax.broadcasted_iota(jnp.int32, (CH, 1), 0)

    def row_chunk(rc, carry):
        rbase = ((r0 + rc * CH) << 12).astype(jnp.uint32)
        M = jnp.full((CH, CW), -jnp.inf, jnp.float32)
        IDX = jnp.zeros((CH, CW), jnp.int32)
        for cc in range(NCC):
            bits = _threefry(RK0, RK1, pat + (rbase + jnp.uint32(cc * CW)),
                             pre_keyed=True)
            f01 = _bits_to_f01(bits)
            u = jnp.maximum(U_MIN, f01 + U_MIN)
            vals = lw_ref[pl.ds(cc * CW, CW)][None, :] + (-jnp.log(-jnp.log(u)))
            upd = vals > M
            M = jnp.where(upd, vals, M)
            IDX = jnp.where(upd, cc, IDX)
        rowV = jnp.max(M, axis=1, keepdims=True)
        jcand = (IDX << 9) | col_s
        ix = jnp.min(jnp.where(M == rowV, jcand, N), axis=1, keepdims=True)
        ix_final = jnp.where(resample, ix, rowid + (r0 + rc * CH))
        ixo_ref[pl.ds(rc * CH, CH), :] = ix_final
        return carry

    jax.lax.fori_loop(0, BLK // CH, row_chunk, 0, unroll=16)


_SC_INFO = plsc.get_sparse_core_info()
_NW = _SC_INFO.num_cores * _SC_INFO.num_subcores
_BPW = N // _NW


def _sc_gather_body(p_hbm, idx_hbm, out_hbm, idx_v, rows_v, sem):
    wid = (jax.lax.axis_index("s") * _SC_INFO.num_cores
           + jax.lax.axis_index("c"))
    base = wid * _BPW
    pltpu.sync_copy(idx_hbm.at[pl.ds(base, _BPW)], idx_v)
    pltpu.async_copy(p_hbm.at[idx_v], rows_v, sem).wait()
    pltpu.sync_copy(rows_v, out_hbm.at[pl.ds(base, _BPW)])


def _sc_gather(particles, idx):
    mesh = plsc.VectorSubcoreMesh(core_axis_name="c", subcore_axis_name="s")
    return pl.kernel(
        _sc_gather_body,
        mesh=mesh,
        out_type=jax.ShapeDtypeStruct((N, D), jnp.float32),
        scratch_types=[
            pltpu.VMEM((_BPW,), jnp.int32),
            pltpu.VMEM((_BPW, D), jnp.float32),
            pltpu.SemaphoreType.DMA,
        ],
    )(particles, idx)


CBLK = 1024
CGRID = N // CBLK


def _epilogue_kernel(lw_ref, pr_ref, obs_ref, logw_ref, next_ref):
    i = pl.program_id(0)
    r0 = i * CBLK

    lw = lw_ref[:]
    m = jnp.max(lw)
    t = jnp.exp(lw - m)
    s1 = jnp.sum(t)
    s2 = jnp.sum(t * t)
    ess = s1 * s1 / (s2 * jnp.float32(N))
    resample = ess < jnp.float32(0.5)

    ctr2 = ((r0 + jax.lax.broadcasted_iota(jnp.int32, (CBLK, D), 0)) * D
            + jax.lax.broadcasted_iota(jnp.int32, (CBLK, D), 1)).astype(jnp.uint32)
    f2 = _bits_to_f01(_threefry(PK0, PK1, ctr2))
    u2 = jnp.maximum(N_LO, f2 * N_SCALE + N_LO)
    seps = jnp.float32(SIGMA) * (SQRT2 * _erfinv(u2))

    pr = pr_ref[...]
    nxt = pr + seps
    next_ref[...] = nxt

    diff = nxt - pr
    obs = obs_ref[:]
    dobs = obs[None, :] - nxt
    half_d = jnp.float32(0.5 * D)
    trans = (-0.5 * jnp.sum((diff / jnp.float32(TAU)) ** 2, axis=1)
             - jnp.float32(D * np.log(TAU)) - half_d * jnp.float32(LOG2PI))
    emis = (-0.5 * jnp.sum((dobs / jnp.float32(R_EMIS)) ** 2, axis=1)
            - jnp.float32(D * np.log(R_EMIS)) - half_d * jnp.float32(LOG2PI))
    prop = (-0.5 * jnp.sum((diff / jnp.float32(SIGMA)) ** 2, axis=1)
            - jnp.float32(D * np.log(SIGMA)) - half_d * jnp.float32(LOG2PI))

    lw_blk = lw_ref[pl.ds(r0, CBLK)]
    base = jnp.where(resample, jnp.float32(0.0), lw_blk)
    logw_ref[...] = base + trans + emis - prop


def kernel(log_weights, particles, observation):
    ixo, ess = pl.pallas_call(
        _argmax_kernel,
        grid=(GRID,),
        in_specs=[pl.BlockSpec((N,), lambda i: (0,))],
        out_specs=[
            pl.BlockSpec((BLK, 1), lambda i: (i, 0)),
            pl.BlockSpec((1, 1, 1), lambda i: (i, 0, 0)),
        ],
        out_shape=[
            jax.ShapeDtypeStruct((N, 1), jnp.int32),
            jax.ShapeDtypeStruct((GRID, 1, 1), jnp.float32),
        ],
        compiler_params=pltpu.CompilerParams(
            dimension_semantics=("parallel",)),
    )(log_weights)

    pr = _sc_gather(particles, ixo.reshape(N))

    logw, nxt = pl.pallas_call(
        _epilogue_kernel,
        grid=(CGRID,),
        in_specs=[
            pl.BlockSpec((N,), lambda i: (0,)),
            pl.BlockSpec((CBLK, D), lambda i: (i, 0)),
            pl.BlockSpec((D,), lambda i: (0,)),
        ],
        out_specs=[
            pl.BlockSpec((CBLK,), lambda i: (i,)),
            pl.BlockSpec((CBLK, D), lambda i: (i, 0)),
        ],
        out_shape=[
            jax.ShapeDtypeStruct((N,), jnp.float32),
            jax.ShapeDtypeStruct((N, D), jnp.float32),
        ],
        compiler_params=pltpu.CompilerParams(
            dimension_semantics=("parallel",)),
    )(log_weights, pr, observation)

    return logw, nxt, ess[0, 0, 0]


# speedup vs baseline: 1.0796x; 1.0068x over previous
import numpy as np
import jax
import jax.numpy as jnp
from jax.experimental import pallas as pl
from jax.experimental.pallas import tpu as pltpu
from jax.experimental.pallas import tpu_sc as plsc

N = 4096
D = 128
TAU = 1.0
SIGMA = 1.2
R_EMIS = 0.5
LOG2PI = float(np.log(2.0 * np.pi))

BLK = 512
GRID = N // BLK


def _np_rotl(x, d):
    return ((x << np.uint32(d)) | (x >> np.uint32(32 - d))).astype(np.uint32)


def _np_threefry2x32(k1, k2, x0, x1):
    x0 = x0.astype(np.uint32)
    x1 = x1.astype(np.uint32)
    ks0 = np.uint32(k1)
    ks1 = np.uint32(k2)
    ks2 = np.uint32(0x1BD11BDA) ^ ks0 ^ ks1
    ks = [ks0, ks1, ks2]
    rots = [(13, 15, 26, 6), (17, 29, 16, 24)]
    x0 = (x0 + ks0).astype(np.uint32)
    x1 = (x1 + ks1).astype(np.uint32)
    for i in range(5):
        for r in rots[i % 2]:
            x0 = (x0 + x1).astype(np.uint32)
            x1 = _np_rotl(x1, r)
            x1 = x1 ^ x0
        x0 = (x0 + ks[(i + 1) % 3]).astype(np.uint32)
        x1 = (x1 + ks[(i + 2) % 3] + np.uint32(i + 1)).astype(np.uint32)
    return x0, x1


_S0, _S1 = _np_threefry2x32(0, 1, np.zeros(2, np.uint32), np.arange(2, dtype=np.uint32))
RK0, RK1 = int(_S0[0]), int(_S1[0])
PK0, PK1 = int(_S0[1]), int(_S1[1])

U_MIN = np.float32(1e-12)
U_SCALE = np.float32(1.0) - np.float32(1e-12)
N_LO = np.float32(np.nextafter(np.float32(-1.0), np.float32(0.0)))
N_SCALE = np.float32(1.0) - N_LO
SQRT2 = np.float32(np.sqrt(2.0))


def _threefry(k1, k2, x1, pre_keyed=False):
    ks0 = jnp.uint32(k1)
    ks1 = jnp.uint32(k2)
    ks2 = jnp.uint32(np.uint32(0x1BD11BDA) ^ np.uint32(k1) ^ np.uint32(k2))
    ks = (ks0, ks1, ks2)
    rots = ((13, 15, 26, 6), (17, 29, 16, 24))
    x0 = jnp.full(x1.shape, ks0, jnp.uint32)
    if not pre_keyed:
        x1 = x1 + ks1
    for i in range(5):
        for r in rots[i % 2]:
            x0 = x0 + x1
            x1 = (x1 << r) | (x1 >> (32 - r))
            x1 = x1 ^ x0
        x0 = x0 + ks[(i + 1) % 3]
        x1 = x1 + ks[(i + 2) % 3] + jnp.uint32(i + 1)
    return x0 ^ x1


def _bits_to_f01(bits):
    fb = (bits >> 9) | jnp.uint32(0x3F800000)
    return jax.lax.bitcast_convert_type(fb, jnp.float32) - jnp.float32(1.0)


def _erfinv(x):
    w = -jnp.log((jnp.float32(1.0) - x) * (jnp.float32(1.0) + x))
    ws = w - jnp.float32(2.5)
    p = jnp.float32(2.81022636e-08)
    for c in (3.43273939e-07, -3.5233877e-06, -4.39150654e-06, 0.00021858087,
              -0.00125372503, -0.00417768164, 0.246640727, 1.50140941):
        p = jnp.float32(c) + p * ws
    wl = jnp.sqrt(w) - jnp.float32(3.0)
    q = jnp.float32(-0.000200214257)
    for c in (0.000100950558, 0.00134934322, -0.00367342844, 0.00573950773,
              -0.0076224613, 0.00943887047, 1.00167406, 2.83297682):
        q = jnp.float32(c) + q * wl
    return jnp.where(w < jnp.float32(5.0), p, q) * x


CH = 8
CW = 512
NCC = N // CW


def _argmax_kernel(lw_ref, ixo_ref, ess_ref):
    i = pl.program_id(0)
    r0 = i * BLK

    lw = lw_ref[:]
    m = jnp.max(lw)
    t = jnp.exp(lw - m)
    s1 = jnp.sum(t)
    s2 = jnp.sum(t * t)
    ess = s1 * s1 / (s2 * jnp.float32(N))
    ess_ref[...] = jnp.reshape(ess, (1, 1, 1))
    resample = ess < jnp.float32(0.5)

    row_s = jax.lax.broadcasted_iota(jnp.int32, (CH, CW), 0)
    col_s = jax.lax.broadcasted_iota(jnp.int32, (CH, CW), 1)
    pat = ((row_s << 12) | col_s).astype(jnp.uint32) + jnp.uint32(RK1)
    rowid = jax.lax.broadcasted_iota(jnp.int32, (CH, 1), 0)

    def row_chunk(rc, carry):
        rbase = ((r0 + rc * CH) << 12).astype(jnp.uint32)
        M = jnp.full((CH, CW), -jnp.inf, jnp.float32)
        IDX = jnp.zeros((CH, CW), jnp.int32)
        for cc in range(NCC):
            bits = _threefry(RK0, RK1, pat + (rbase + jnp.uint32(cc * CW)),
                             pre_keyed=True)
            f01 = _bits_to_f01(bits)
            u = jnp.maximum(U_MIN, f01 + U_MIN)
            vals = lw_ref[pl.ds(cc * CW, CW)][None, :] + (-jnp.log(-jnp.log(u)))
            upd = vals > M
            M = jnp.where(upd, vals, M)
            IDX = jnp.where(upd, cc, IDX)
        rowV = jnp.max(M, axis=1, keepdims=True)
        jcand = (IDX << 9) | col_s
        ix = jnp.min(jnp.where(M == rowV, jcand, N), axis=1, keepdims=True)
        ix_final = jnp.where(resample, ix, rowid + (r0 + rc * CH))
        ixo_ref[pl.ds(rc * CH, CH), :] = ix_final
        return carry

    jax.lax.fori_loop(0, BLK // CH, row_chunk, 0, unroll=32)


_SC_INFO = plsc.get_sparse_core_info()
_NW = _SC_INFO.num_cores * _SC_INFO.num_subcores
_BPW = N // _NW


def _sc_gather_body(p_hbm, idx_hbm, out_hbm, idx_v, rows_v, sem):
    wid = (jax.lax.axis_index("s") * _SC_INFO.num_cores
           + jax.lax.axis_index("c"))
    base = wid * _BPW
    pltpu.sync_copy(idx_hbm.at[pl.ds(base, _BPW)], idx_v)
    pltpu.async_copy(p_hbm.at[idx_v], rows_v, sem).wait()
    pltpu.sync_copy(rows_v, out_hbm.at[pl.ds(base, _BPW)])


def _sc_gather(particles, idx):
    mesh = plsc.VectorSubcoreMesh(core_axis_name="c", subcore_axis_name="s")
    return pl.kernel(
        _sc_gather_body,
        mesh=mesh,
        out_type=jax.ShapeDtypeStruct((N, D), jnp.float32),
        scratch_types=[
            pltpu.VMEM((_BPW,), jnp.int32),
            pltpu.VMEM((_BPW, D), jnp.float32),
            pltpu.SemaphoreType.DMA,
        ],
    )(particles, idx)


CBLK = 1024
CGRID = N // CBLK


def _epilogue_kernel(lw_ref, pr_ref, obs_ref, logw_ref, next_ref):
    i = pl.program_id(0)
    r0 = i * CBLK

    lw = lw_ref[:]
    m = jnp.max(lw)
    t = jnp.exp(lw - m)
    s1 = jnp.sum(t)
    s2 = jnp.sum(t * t)
    ess = s1 * s1 / (s2 * jnp.float32(N))
    resample = ess < jnp.float32(0.5)

    ctr2 = ((r0 + jax.lax.broadcasted_iota(jnp.int32, (CBLK, D), 0)) * D
            + jax.lax.broadcasted_iota(jnp.int32, (CBLK, D), 1)).astype(jnp.uint32)
    f2 = _bits_to_f01(_threefry(PK0, PK1, ctr2))
    u2 = jnp.maximum(N_LO, f2 * N_SCALE + N_LO)
    seps = jnp.float32(SIGMA) * (SQRT2 * _erfinv(u2))

    pr = pr_ref[...]
    nxt = pr + seps
    next_ref[...] = nxt

    diff = nxt - pr
    obs = obs_ref[:]
    dobs = obs[None, :] - nxt
    half_d = jnp.float32(0.5 * D)
    trans = (-0.5 * jnp.sum((diff / jnp.float32(TAU)) ** 2, axis=1)
             - jnp.float32(D * np.log(TAU)) - half_d * jnp.float32(LOG2PI))
    emis = (-0.5 * jnp.sum((dobs / jnp.float32(R_EMIS)) ** 2, axis=1)
            - jnp.float32(D * np.log(R_EMIS)) - half_d * jnp.float32(LOG2PI))
    prop = (-0.5 * jnp.sum((diff / jnp.float32(SIGMA)) ** 2, axis=1)
            - jnp.float32(D * np.log(SIGMA)) - half_d * jnp.float32(LOG2PI))

    lw_blk = lw_ref[pl.ds(r0, CBLK)]
    base = jnp.where(resample, jnp.float32(0.0), lw_blk)
    logw_ref[...] = base + trans + emis - prop


def kernel(log_weights, particles, observation):
    ixo, ess = pl.pallas_call(
        _argmax_kernel,
        grid=(GRID,),
        in_specs=[pl.BlockSpec((N,), lambda i: (0,))],
        out_specs=[
            pl.BlockSpec((BLK, 1), lambda i: (i, 0)),
            pl.BlockSpec((1, 1, 1), lambda i: (i, 0, 0)),
        ],
        out_shape=[
            jax.ShapeDtypeStruct((N, 1), jnp.int32),
            jax.ShapeDtypeStruct((GRID, 1, 1), jnp.float32),
        ],
        compiler_params=pltpu.CompilerParams(
            dimension_semantics=("parallel",)),
    )(log_weights)

    pr = _sc_gather(particles, ixo.reshape(N))

    logw, nxt = pl.pallas_call(
        _epilogue_kernel,
        grid=(CGRID,),
        in_specs=[
            pl.BlockSpec((N,), lambda i: (0,)),
            pl.BlockSpec((CBLK, D), lambda i: (i, 0)),
            pl.BlockSpec((D,), lambda i: (0,)),
        ],
        out_specs=[
            pl.BlockSpec((CBLK,), lambda i: (i,)),
            pl.BlockSpec((CBLK, D), lambda i: (i, 0)),
        ],
        out_shape=[
            jax.ShapeDtypeStruct((N,), jnp.float32),
            jax.ShapeDtypeStruct((N, D), jnp.float32),
        ],
        compiler_params=pltpu.CompilerParams(
            dimension_semantics=("parallel",)),
    )(log_weights, pr, observation)

    return logw, nxt, ess[0, 0, 0]
